# baseline (speedup 1.0000x reference)
import functools

import jax
import jax.numpy as jnp
from jax import lax
from jax.experimental import pallas as pl
from jax.experimental.pallas import tpu as pltpu
from jax.experimental.pallas import tpu_sc as plsc

B = 16384
V = 1000000
D = 32
F = 53
NC = 2
NS = 16
NW = NC * NS
BPW = B // NW
C = 32
NCHUNK = BPW // C
CF = C * F

NBLK = V // 128
NTAIL = V - NBLK * 128
SB = 512
NSB = NBLK * 128 // SB
K1_ITERS = (NSB + NW - 1) // NW

_mesh = plsc.VectorSubcoreMesh(core_axis_name="c", subcore_axis_name="s")


def _t2r_body(tt_hbm, tail_hbm, out_hbm,
              buf0, buf1, obuf,
              isem0, isem1):
  wid = lax.axis_index("s") * NC + lax.axis_index("c")
  iota = lax.iota(jnp.int32, 16)
  iohi = iota + 16
  bufs = (buf0, buf1)
  isems = (isem0, isem1)

  def blk(i):
    return jnp.minimum(i * NW + wid, NSB - 1)

  def fire_in(b, par):
    for fb in range(4):
      pltpu.async_copy(tt_hbm.at[pl.ds(fb * 8, 8), pl.ds(b * SB, SB)],
                       bufs[par].at[pl.ds(fb * 8, 8), pl.ds(0, SB)],
                       isems[par])

  def wait_in(par):
    for fb in range(4):
      pltpu.make_async_copy(tt_hbm.at[pl.ds(fb * 8, 8), pl.ds(0, SB)],
                            bufs[par].at[pl.ds(fb * 8, 8), pl.ds(0, SB)],
                            isems[par]).wait()

  fire_in(blk(0), 0)
  fire_in(blk(1), 1)

  zero16 = jnp.zeros((16,), jnp.int32)

  def process(i, par):
    b = blk(i)
    wait_in(par)
    buf = bufs[par]

    def rbody(r4, rv):
      for u in range(4):
        r = r4 * 4 + u
        lo = plsc.load_gather(buf, [iota, rv])
        hi = plsc.load_gather(buf, [iohi, rv])
        obuf[pl.ds(r * D, 16)] = lo
        obuf[pl.ds(r * D + 16, 16)] = hi
        rv = rv + 1
      return rv

    lax.fori_loop(0, SB // 4, rbody, zero16)

    @pl.when(i + 2 < K1_ITERS)
    def _():
      fire_in(blk(i + 2), par)

    pltpu.sync_copy(obuf, out_hbm.at[pl.ds(b * SB * D, SB * D)])

  def step_body(step, _):
    for par in range(2):
      process(step * 2 + par, par)
    return 0

  lax.fori_loop(0, K1_ITERS // 2, step_body, 0)

  @pl.when(wid == 0)
  def _():
    pltpu.sync_copy(tail_hbm, out_hbm.at[pl.ds(NBLK * 128 * D, NTAIL * D)])


_t2r_kernel = pl.kernel(
    _t2r_body,
    out_type=jax.ShapeDtypeStruct((V * D,), jnp.float32),
    mesh=_mesh,
    compiler_params=pltpu.CompilerParams(use_tc_tiling_on_sc=True,
                                         needs_layout_passes=False),
    scratch_types=[
        pltpu.VMEM((32, SB + 1), jnp.float32),
        pltpu.VMEM((32, SB + 1), jnp.float32),
        pltpu.VMEM((SB * D,), jnp.float32),
        pltpu.SemaphoreType.DMA,
        pltpu.SemaphoreType.DMA,
    ],
)


def _fm_body(idx_hbm, table_hbm, w_hbm, bias_hbm,
             logit_hbm, user_hbm, item_hbm,
             idx_v0, idx_v1, rows_v0, rows_v1, w_v0, w_v1,
             uo_v, io_v, lo_v, so_v, bias_v,
             isem0, isem1, gsem0, gsem1):
  wid = lax.axis_index("s") * NC + lax.axis_index("c")
  ebase = wid * BPW

  idx_vs = (idx_v0, idx_v1)
  rows_vs = (rows_v0, rows_v1)
  w_vs = (w_v0, w_v1)
  isems = (isem0, isem1)
  gsems = (gsem0, gsem1)

  pltpu.sync_copy(bias_hbm, bias_v)

  def idx_src(c):
    return idx_hbm.at[pl.ds((ebase + c * C) * F, CF)]

  def fire_gather(par):
    pltpu.async_copy(table_hbm.at[idx_vs[par]], rows_vs[par], gsems[par])
    pltpu.async_copy(w_hbm.at[idx_vs[par]], w_vs[par].at[pl.ds(0, CF)],
                     gsems[par])

  def wait_idx(par):
    pltpu.make_async_copy(idx_src(0), idx_vs[par], isems[par]).wait()

  def wait_gather(par):
    pltpu.make_async_copy(table_hbm.at[idx_vs[par]], rows_vs[par],
                          gsems[par]).wait()
    pltpu.make_async_copy(w_hbm.at[idx_vs[par]], w_vs[par].at[pl.ds(0, CF)],
                          gsems[par]).wait()

  pltpu.async_copy(idx_src(0), idx_vs[0], isems[0])
  pltpu.async_copy(idx_src(1), idx_vs[1], isems[1])
  wait_idx(0)
  fire_gather(0)

  iota = lax.iota(jnp.int32, 16)
  wmask = iota < (F - 48)
  perms = [jnp.bitwise_xor(iota, sh) for sh in (1, 2, 4, 8)]
  zero = jnp.zeros((16,), jnp.float32)

  _dnums = lax.GatherDimensionNumbers(
      offset_dims=(), collapsed_slice_dims=(0,), start_index_map=(0,))

  def xlane_sum(x):
    for p in perms:
      x = x + lax.gather(x, p.reshape(16, 1), _dnums, (1,),
                         mode=lax.GatherScatterMode.PROMISE_IN_BOUNDS)
    return x

  def compute(c, par):
    rows = rows_vs[par]
    wv = w_vs[par]

    def ebody(e, acc):
      r = e * F

      def accum(j, u0, u1, q0, q1):
        v0 = rows[r + j, pl.ds(0, 16)]
        v1 = rows[r + j, pl.ds(16, 16)]
        return u0 + v0, u1 + v1, q0 + v0 * v0, q1 + v1 * v1

      u0 = u1 = q0 = q1 = zero
      for j in range(51):
        u0, u1, q0, q1 = accum(j, u0, u1, q0, q1)
      i0 = i1 = zero
      for j in range(51, 53):
        i0, i1, q0, q1 = accum(j, i0, i1, q0, q1)

      s0 = u0 + i0
      s1 = u1 + i1
      t = (s0 * s0 - q0) + (s1 * s1 - q1)

      w0 = wv[pl.ds(r, 16)]
      w1 = wv[pl.ds(r + 16, 16)]
      w2 = wv[pl.ds(r + 32, 16)]
      w3 = jnp.where(wmask, wv[pl.ds(r + 48, 16)], 0.0)

      x = xlane_sum(0.5 * t + ((w0 + w1) + (w2 + w3)))
      acc = jnp.where(iota == (e & 15), x, acc)

      @pl.when((e & 15) == 15)
      def _():
        so_v[pl.ds(e - 15, 16)] = acc

      uo_v[pl.ds(e * D, 16)] = u0
      uo_v[pl.ds(e * D + 16, 16)] = u1
      io_v[pl.ds(e * D, 16)] = i0
      io_v[pl.ds(e * D + 16, 16)] = i1
      return acc

    lax.fori_loop(0, C, ebody, zero)

    bvec = bias_v[pl.ds(0, 16)]
    for k in range(C // 16):
      xv = so_v[pl.ds(k * 16, 16)] + bvec
      lo_v[pl.ds(k * 16, 16)] = 1.0 / (1.0 + jnp.exp(-xv))

    obase = ebase + c * C
    pltpu.sync_copy(uo_v, user_hbm.at[pl.ds(obase * D, C * D)])
    pltpu.sync_copy(io_v, item_hbm.at[pl.ds(obase * D, C * D)])
    pltpu.sync_copy(lo_v, logit_hbm.at[pl.ds(obase, C)])

  def step_body(step, _):
    for par in range(2):
      c = step * 2 + par

      @pl.when(c + 1 < NCHUNK)
      def _():
        wait_idx(1 - par)
        fire_gather(1 - par)

      wait_gather(par)

      @pl.when(c + 2 < NCHUNK)
      def _():
        pltpu.async_copy(idx_src(c + 2), idx_vs[par], isems[par])

      compute(c, par)
    return 0

  lax.fori_loop(0, NCHUNK // 2, step_body, 0)


_fm_kernel = pl.kernel(
    _fm_body,
    out_type=(
        jax.ShapeDtypeStruct((B,), jnp.float32),
        jax.ShapeDtypeStruct((B * D,), jnp.float32),
        jax.ShapeDtypeStruct((B * D,), jnp.float32),
    ),
    mesh=_mesh,
    compiler_params=pltpu.CompilerParams(use_tc_tiling_on_sc=False),
    scratch_types=[
        pltpu.VMEM((CF,), jnp.int32),
        pltpu.VMEM((CF,), jnp.int32),
        pltpu.VMEM((CF, D), jnp.float32),
        pltpu.VMEM((CF, D), jnp.float32),
        pltpu.VMEM((CF + 16,), jnp.float32),
        pltpu.VMEM((CF + 16,), jnp.float32),
        pltpu.VMEM((C * D,), jnp.float32),
        pltpu.VMEM((C * D,), jnp.float32),
        pltpu.VMEM((C,), jnp.float32),
        pltpu.VMEM((C,), jnp.float32),
        pltpu.VMEM((16,), jnp.float32),
        pltpu.SemaphoreType.DMA,
        pltpu.SemaphoreType.DMA,
        pltpu.SemaphoreType.DMA,
        pltpu.SemaphoreType.DMA,
    ],
)


@jax.jit
def _run(user_id, user_hist, item_id, item_cat, embed_table, w_table, bias):
  idx = jnp.concatenate(
      [user_id, user_hist, item_id, item_cat], axis=1
  ).astype(jnp.int32).reshape(B * F)
  w_flat = w_table.reshape(V)
  bias16 = jnp.broadcast_to(bias.astype(jnp.float32), (16,))
  tail = embed_table[NBLK * 128:].reshape(NTAIL * D)
  table_lin = _t2r_kernel(embed_table.T, tail).reshape(V, D)
  logit, user_flat, item_flat = _fm_kernel(idx, table_lin, w_flat, bias16)
  return (logit.reshape(B, 1), user_flat.reshape(B, D),
          item_flat.reshape(B, D))


def kernel(user_id, user_hist, item_id, item_cat, embed_table, w_table, bias):
  return _run(user_id, user_hist, item_id, item_cat, embed_table, w_table,
              bias)


# speedup vs baseline: 1.3733x; 1.3733x over previous
import functools

import jax
import jax.numpy as jnp
from jax import lax
from jax.experimental import pallas as pl
from jax.experimental.pallas import tpu as pltpu
from jax.experimental.pallas import tpu_sc as plsc

B = 16384
V = 1000000
D = 32
F = 53
NC = 2
NS = 16
NW = NC * NS
BPW = B // NW
C = 32
NCHUNK = BPW // C
CF = C * F

NBLK = V // 128
NTAIL = V - NBLK * 128
SB = 512
NSB = NBLK * 128 // SB
K1_ITERS = (NSB + NW - 1) // NW

_mesh = plsc.VectorSubcoreMesh(core_axis_name="c", subcore_axis_name="s")


def _fm_body(idx_hbm, table_hbm, w_hbm, bias_hbm,
             logit_hbm, user_hbm, item_hbm,
             idx_v0, idx_v1, rows_v0, rows_v1, w_v0, w_v1,
             uo_v, io_v, lo_v, so_v, bias_v,
             isem0, isem1, gsem0, gsem1):
  wid = lax.axis_index("s") * NC + lax.axis_index("c")
  ebase = wid * BPW

  idx_vs = (idx_v0, idx_v1)
  rows_vs = (rows_v0, rows_v1)
  w_vs = (w_v0, w_v1)
  isems = (isem0, isem1)
  gsems = (gsem0, gsem1)

  pltpu.sync_copy(bias_hbm, bias_v)

  def idx_src(c):
    return idx_hbm.at[pl.ds((ebase + c * C) * F, CF)]

  def fire_gather(par):
    pltpu.async_copy(table_hbm.at[idx_vs[par]], rows_vs[par], gsems[par])
    pltpu.async_copy(w_hbm.at[idx_vs[par]], w_vs[par].at[pl.ds(0, CF)],
                     gsems[par])

  def wait_idx(par):
    pltpu.make_async_copy(idx_src(0), idx_vs[par], isems[par]).wait()

  def wait_gather(par):
    pltpu.make_async_copy(table_hbm.at[idx_vs[par]], rows_vs[par],
                          gsems[par]).wait()
    pltpu.make_async_copy(w_hbm.at[idx_vs[par]], w_vs[par].at[pl.ds(0, CF)],
                          gsems[par]).wait()

  pltpu.async_copy(idx_src(0), idx_vs[0], isems[0])
  pltpu.async_copy(idx_src(1), idx_vs[1], isems[1])
  wait_idx(0)
  fire_gather(0)

  iota = lax.iota(jnp.int32, 16)
  wmask = iota < (F - 48)
  perms = [jnp.bitwise_xor(iota, sh) for sh in (1, 2, 4, 8)]
  zero = jnp.zeros((16,), jnp.float32)

  _dnums = lax.GatherDimensionNumbers(
      offset_dims=(), collapsed_slice_dims=(0,), start_index_map=(0,))

  def xlane_sum(x):
    for p in perms:
      x = x + lax.gather(x, p.reshape(16, 1), _dnums, (1,),
                         mode=lax.GatherScatterMode.PROMISE_IN_BOUNDS)
    return x

  def compute(c, par):
    rows = rows_vs[par]
    wv = w_vs[par]

    def ebody(e, acc):
      r = e * F

      def accum(j, u0, u1, q0, q1):
        v0 = rows[r + j, pl.ds(0, 16)]
        v1 = rows[r + j, pl.ds(16, 16)]
        return u0 + v0, u1 + v1, q0 + v0 * v0, q1 + v1 * v1

      u0 = u1 = q0 = q1 = zero
      for j in range(51):
        u0, u1, q0, q1 = accum(j, u0, u1, q0, q1)
      i0 = i1 = zero
      for j in range(51, 53):
        i0, i1, q0, q1 = accum(j, i0, i1, q0, q1)

      s0 = u0 + i0
      s1 = u1 + i1
      t = (s0 * s0 - q0) + (s1 * s1 - q1)

      w0 = wv[pl.ds(r, 16)]
      w1 = wv[pl.ds(r + 16, 16)]
      w2 = wv[pl.ds(r + 32, 16)]
      w3 = jnp.where(wmask, wv[pl.ds(r + 48, 16)], 0.0)

      x = xlane_sum(0.5 * t + ((w0 + w1) + (w2 + w3)))
      acc = jnp.where(iota == (e & 15), x, acc)

      @pl.when((e & 15) == 15)
      def _():
        so_v[pl.ds(e - 15, 16)] = acc

      uo_v[pl.ds(e * D, 16)] = u0
      uo_v[pl.ds(e * D + 16, 16)] = u1
      io_v[pl.ds(e * D, 16)] = i0
      io_v[pl.ds(e * D + 16, 16)] = i1
      return acc

    lax.fori_loop(0, C, ebody, zero)

    bvec = bias_v[pl.ds(0, 16)]
    for k in range(C // 16):
      xv = so_v[pl.ds(k * 16, 16)] + bvec
      lo_v[pl.ds(k * 16, 16)] = 1.0 / (1.0 + jnp.exp(-xv))

    obase = ebase + c * C
    pltpu.sync_copy(uo_v, user_hbm.at[pl.ds(obase * D, C * D)])
    pltpu.sync_copy(io_v, item_hbm.at[pl.ds(obase * D, C * D)])
    pltpu.sync_copy(lo_v, logit_hbm.at[pl.ds(obase, C)])

  def step_body(step, _):
    for par in range(2):
      c = step * 2 + par

      @pl.when(c + 1 < NCHUNK)
      def _():
        wait_idx(1 - par)
        fire_gather(1 - par)

      wait_gather(par)

      @pl.when(c + 2 < NCHUNK)
      def _():
        pltpu.async_copy(idx_src(c + 2), idx_vs[par], isems[par])

      compute(c, par)
    return 0

  lax.fori_loop(0, NCHUNK // 2, step_body, 0)


_fm_kernel = pl.kernel(
    _fm_body,
    out_type=(
        jax.ShapeDtypeStruct((B,), jnp.float32),
        jax.ShapeDtypeStruct((B * D,), jnp.float32),
        jax.ShapeDtypeStruct((B * D,), jnp.float32),
    ),
    mesh=_mesh,
    compiler_params=pltpu.CompilerParams(use_tc_tiling_on_sc=False),
    scratch_types=[
        pltpu.VMEM((CF,), jnp.int32),
        pltpu.VMEM((CF,), jnp.int32),
        pltpu.VMEM((CF, D), jnp.float32),
        pltpu.VMEM((CF, D), jnp.float32),
        pltpu.VMEM((CF + 16,), jnp.float32),
        pltpu.VMEM((CF + 16,), jnp.float32),
        pltpu.VMEM((C * D,), jnp.float32),
        pltpu.VMEM((C * D,), jnp.float32),
        pltpu.VMEM((C,), jnp.float32),
        pltpu.VMEM((C,), jnp.float32),
        pltpu.VMEM((16,), jnp.float32),
        pltpu.SemaphoreType.DMA,
        pltpu.SemaphoreType.DMA,
        pltpu.SemaphoreType.DMA,
        pltpu.SemaphoreType.DMA,
    ],
)


@jax.jit
def _run(user_id, user_hist, item_id, item_cat, embed_table, w_table, bias):
  idx = jnp.concatenate(
      [user_id, user_hist, item_id, item_cat], axis=1
  ).astype(jnp.int32).reshape(B * F)
  w_flat = w_table.reshape(V)
  bias16 = jnp.broadcast_to(bias.astype(jnp.float32), (16,))
  table_lin = lax.optimization_barrier(
      embed_table.reshape(V * D)).reshape(V, D)
  logit, user_flat, item_flat = _fm_kernel(idx, table_lin, w_flat, bias16)
  return (logit.reshape(B, 1), user_flat.reshape(B, D),
          item_flat.reshape(B, D))


def kernel(user_id, user_hist, item_id, item_cat, embed_table, w_table, bias):
  return _run(user_id, user_hist, item_id, item_cat, embed_table, w_table,
              bias)
